# speedup vs baseline: 1.0372x; 1.0372x over previous
import functools

import numpy as np
import jax
import jax.numpy as jnp
from jax.experimental import pallas as pl
from jax.experimental.pallas import tpu as pltpu


def _lin_interp(n_in, n_out):
    if n_in == 1:
        return np.ones((n_out, 1), np.float32)
    pos = np.arange(n_out, dtype=np.float64) * (n_in - 1) / (n_out - 1)
    left = np.minimum(pos.astype(np.int64), n_in - 2)
    frac = pos - left
    m = np.zeros((n_out, n_in), np.float64)
    m[np.arange(n_out), left] = 1.0 - frac
    m[np.arange(n_out), left + 1] += frac
    return m.astype(np.float32)


@functools.lru_cache(maxsize=None)
def _up2_kron(h, w):
    kh = _lin_interp(h, 2 * h).T
    kw = _lin_interp(w, 2 * w).T
    return np.kron(kh, kw)


def _fpn_body(c5_ref, c4_ref, c3_ref, w4_ref, b4_ref, w3_ref, b3_ref,
              k1_ref, k2_ref, p3_ref, p4_ref, p5_ref):
    bf16, f32 = jnp.bfloat16, jnp.float32
    x5 = c5_ref[0].astype(bf16)
    y5 = jnp.dot(w4_ref[...], x5, preferred_element_type=f32) + b4_ref[...]
    p5 = jnp.dot(y5.astype(bf16), k1_ref[...],
                 preferred_element_type=f32)
    p5_ref[0] = p5
    t = (p5 + c4_ref[0]).astype(bf16)
    y4 = jnp.dot(w3_ref[...], t, preferred_element_type=f32) + b3_ref[...]
    p4 = jnp.dot(y4.astype(bf16), k2_ref[...],
                 preferred_element_type=f32)
    p4_ref[0] = p4
    p3_ref[0] = p4 + c3_ref[0]


def kernel(w_c4, b_c4, w_c3, b_c3, c3, c4, c5):
    B, C5, H5, W5 = c5.shape
    C4 = w_c4.shape[0]
    C3 = w_c3.shape[0]
    HW5 = H5 * W5
    HW4 = 4 * HW5
    HW3 = 16 * HW5
    bf16 = jnp.bfloat16

    k1 = jnp.asarray(_up2_kron(H5, W5), dtype=bf16)
    k2 = jnp.asarray(_up2_kron(2 * H5, 2 * W5), dtype=bf16)

    operands = (
        c5.reshape(B, C5, HW5),
        c4.reshape(B, C4, HW4),
        c3.reshape(B, C3, HW3),
        w_c4.astype(bf16), b_c4.reshape(C4, 1),
        w_c3.astype(bf16), b_c3.reshape(C3, 1),
        k1, k2,
    )
    batch_spec = lambda shape: pl.BlockSpec((1,) + shape, lambda i: (i, 0, 0))
    fixed_spec = lambda shape: pl.BlockSpec(shape, lambda i: (0, 0))
    in_specs = [
        batch_spec((C5, HW5)),
        batch_spec((C4, HW4)),
        batch_spec((C3, HW3)),
        fixed_spec((C4, C5)), fixed_spec((C4, 1)),
        fixed_spec((C3, C4)), fixed_spec((C3, 1)),
        fixed_spec((HW5, HW4)), fixed_spec((HW4, HW3)),
    ]
    p3, p4, p5 = pl.pallas_call(
        _fpn_body,
        grid=(B,),
        in_specs=in_specs,
        out_specs=(batch_spec((C3, HW3)), batch_spec((C3, HW3)),
                   batch_spec((C4, HW4))),
        out_shape=(jax.ShapeDtypeStruct((B, C3, HW3), c3.dtype),
                   jax.ShapeDtypeStruct((B, C3, HW3), c3.dtype),
                   jax.ShapeDtypeStruct((B, C4, HW4), c4.dtype)),
        compiler_params=pltpu.CompilerParams(
            dimension_semantics=("parallel",),
            vmem_limit_bytes=64 << 20),
    )(*operands)
    return (p3.reshape(B, C3, 4 * H5, 4 * W5),
            p4.reshape(B, C3, 4 * H5, 4 * W5),
            p5.reshape(B, C4, 2 * H5, 2 * W5))


# speedup vs baseline: 1.1082x; 1.0684x over previous
import functools

import numpy as np
import jax
import jax.numpy as jnp
from jax.experimental import pallas as pl
from jax.experimental.pallas import tpu as pltpu


def _lin_interp(n_in, n_out):
    if n_in == 1:
        return np.ones((n_out, 1), np.float32)
    pos = np.arange(n_out, dtype=np.float64) * (n_in - 1) / (n_out - 1)
    left = np.minimum(pos.astype(np.int64), n_in - 2)
    frac = pos - left
    m = np.zeros((n_out, n_in), np.float64)
    m[np.arange(n_out), left] = 1.0 - frac
    m[np.arange(n_out), left + 1] += frac
    return m.astype(np.float32)


@functools.lru_cache(maxsize=None)
def _up2_kron(h, w):
    kh = _lin_interp(h, 2 * h).T
    kw = _lin_interp(w, 2 * w).T
    return np.kron(kh, kw)


def _fpn_body(c5_ref, c4_ref, c3_ref, w4_ref, b4_ref, w3_ref, b3_ref,
              k1_ref, k2_ref, p3_ref, p4_ref, p5_ref):
    bf16, f32 = jnp.bfloat16, jnp.float32
    for g in range(c5_ref.shape[0]):
        x5 = c5_ref[g].astype(bf16)
        y5 = jnp.dot(w4_ref[...], x5,
                     preferred_element_type=f32) + b4_ref[...]
        p5 = jnp.dot(y5.astype(bf16), k1_ref[...],
                     preferred_element_type=f32)
        p5_ref[g] = p5
        t = (p5 + c4_ref[g]).astype(bf16)
        y4 = jnp.dot(w3_ref[...], t,
                     preferred_element_type=f32) + b3_ref[...]
        p4 = jnp.dot(y4.astype(bf16), k2_ref[...],
                     preferred_element_type=f32)
        p4_ref[g] = p4
        p3_ref[g] = p4 + c3_ref[g]


def kernel(w_c4, b_c4, w_c3, b_c3, c3, c4, c5):
    B, C5, H5, W5 = c5.shape
    C4 = w_c4.shape[0]
    C3 = w_c3.shape[0]
    HW5 = H5 * W5
    HW4 = 4 * HW5
    HW3 = 16 * HW5
    bf16 = jnp.bfloat16

    k1 = jnp.asarray(_up2_kron(H5, W5), dtype=bf16)
    k2 = jnp.asarray(_up2_kron(2 * H5, 2 * W5), dtype=bf16)

    operands = (
        c5.reshape(B, C5, HW5),
        c4.reshape(B, C4, HW4),
        c3.reshape(B, C3, HW3),
        w_c4.astype(bf16), b_c4.reshape(C4, 1),
        w_c3.astype(bf16), b_c3.reshape(C3, 1),
        k1, k2,
    )
    G = 4 if B % 4 == 0 else (2 if B % 2 == 0 else 1)
    batch_spec = lambda shape: pl.BlockSpec((G,) + shape, lambda i: (i, 0, 0))
    fixed_spec = lambda shape: pl.BlockSpec(shape, lambda i: (0, 0))
    in_specs = [
        batch_spec((C5, HW5)),
        batch_spec((C4, HW4)),
        batch_spec((C3, HW3)),
        fixed_spec((C4, C5)), fixed_spec((C4, 1)),
        fixed_spec((C3, C4)), fixed_spec((C3, 1)),
        fixed_spec((HW5, HW4)), fixed_spec((HW4, HW3)),
    ]
    p3, p4, p5 = pl.pallas_call(
        _fpn_body,
        grid=(B // G,),
        in_specs=in_specs,
        out_specs=(batch_spec((C3, HW3)), batch_spec((C3, HW3)),
                   batch_spec((C4, HW4))),
        out_shape=(jax.ShapeDtypeStruct((B, C3, HW3), c3.dtype),
                   jax.ShapeDtypeStruct((B, C3, HW3), c3.dtype),
                   jax.ShapeDtypeStruct((B, C4, HW4), c4.dtype)),
        compiler_params=pltpu.CompilerParams(
            dimension_semantics=("parallel",),
            vmem_limit_bytes=64 << 20),
    )(*operands)
    return (p3.reshape(B, C3, 4 * H5, 4 * W5),
            p4.reshape(B, C3, 4 * H5, 4 * W5),
            p5.reshape(B, C4, 2 * H5, 2 * W5))


# speedup vs baseline: 4.1432x; 3.7387x over previous
import functools

import numpy as np
import jax
import jax.numpy as jnp
from jax.experimental import pallas as pl
from jax.experimental.pallas import tpu as pltpu


def _lin_interp(n_in, n_out):
    if n_in == 1:
        return np.ones((n_out, 1), np.float32)
    pos = np.arange(n_out, dtype=np.float64) * (n_in - 1) / (n_out - 1)
    left = np.minimum(pos.astype(np.int64), n_in - 2)
    frac = pos - left
    m = np.zeros((n_out, n_in), np.float64)
    m[np.arange(n_out), left] = 1.0 - frac
    m[np.arange(n_out), left + 1] += frac
    return m.astype(np.float32)


@functools.lru_cache(maxsize=None)
def _up2_matrix(h, w):
    return np.kron(_lin_interp(h, 2 * h), _lin_interp(w, 2 * w))


def _fpn_body(c5_ref, c4_ref, c3_ref, w4t_ref, b4_ref, w3t_ref, b3_ref,
              u1_ref, u2_ref, p3_ref, p4_ref, p5_ref):
    bf16, f32 = jnp.bfloat16, jnp.float32
    G, HW5, C5 = c5_ref.shape
    x5 = c5_ref[...].reshape(G * HW5, C5).astype(bf16)
    y5 = jnp.dot(x5, w4t_ref[...], preferred_element_type=f32) + b4_ref[...]
    y5 = y5.astype(bf16).reshape(G, HW5, -1)
    for g in range(G):
        p5 = jnp.dot(u1_ref[...], y5[g], preferred_element_type=f32)
        p5_ref[g] = p5
        t = (p5 + c4_ref[g]).astype(bf16)
        y4 = jnp.dot(t, w3t_ref[...], preferred_element_type=f32) + b3_ref[...]
        p4 = jnp.dot(u2_ref[...], y4.astype(bf16), preferred_element_type=f32)
        p4_ref[g] = p4
        p3_ref[g] = p4 + c3_ref[g]


def kernel(w_c4, b_c4, w_c3, b_c3, c3, c4, c5):
    B, C5, H5, W5 = c5.shape
    C4 = w_c4.shape[0]
    C3 = w_c3.shape[0]
    HW5 = H5 * W5
    HW4 = 4 * HW5
    HW3 = 16 * HW5
    bf16 = jnp.bfloat16

    u1 = jnp.asarray(_up2_matrix(H5, W5), dtype=bf16)
    u2 = jnp.asarray(_up2_matrix(2 * H5, 2 * W5), dtype=bf16)

    c5r = jnp.transpose(c5, (0, 2, 3, 1)).reshape(B, HW5, C5)
    c4r = jnp.transpose(c4, (0, 2, 3, 1)).reshape(B, HW4, C4)
    c3r = jnp.transpose(c3, (0, 2, 3, 1)).reshape(B, HW3, C3)

    G = 4 if B % 4 == 0 else (2 if B % 2 == 0 else 1)
    batch_spec = lambda r, c: pl.BlockSpec((G, r, c), lambda i: (i, 0, 0))
    fixed_spec = lambda r, c: pl.BlockSpec((r, c), lambda i: (0, 0))
    in_specs = [
        batch_spec(HW5, C5),
        batch_spec(HW4, C4),
        batch_spec(HW3, C3),
        fixed_spec(C5, C4), fixed_spec(1, C4),
        fixed_spec(C4, C3), fixed_spec(1, C3),
        fixed_spec(HW4, HW5), fixed_spec(HW3, HW4),
    ]
    p3, p4, p5 = pl.pallas_call(
        _fpn_body,
        grid=(B // G,),
        in_specs=in_specs,
        out_specs=(batch_spec(HW3, C3), batch_spec(HW3, C3),
                   batch_spec(HW4, C4)),
        out_shape=(jax.ShapeDtypeStruct((B, HW3, C3), c3.dtype),
                   jax.ShapeDtypeStruct((B, HW3, C3), c3.dtype),
                   jax.ShapeDtypeStruct((B, HW4, C4), c4.dtype)),
        compiler_params=pltpu.CompilerParams(
            dimension_semantics=("parallel",),
            vmem_limit_bytes=64 << 20),
    )(c5r, c4r, c3r,
      w_c4.T.astype(bf16), b_c4.reshape(1, C4),
      w_c3.T.astype(bf16), b_c3.reshape(1, C3),
      u1, u2)

    def _to_nchw(x, c, hw):
        h = int(round(hw ** 0.5))
        return jnp.transpose(x.reshape(B, h, h, c), (0, 3, 1, 2))

    return (_to_nchw(p3, C3, HW3), _to_nchw(p4, C3, HW3),
            _to_nchw(p5, C4, HW4))


# speedup vs baseline: 4.3063x; 1.0394x over previous
import functools

import numpy as np
import jax
import jax.numpy as jnp
from jax.experimental import pallas as pl
from jax.experimental.pallas import tpu as pltpu


def _lin_interp(n_in, n_out):
    if n_in == 1:
        return np.ones((n_out, 1), np.float32)
    pos = np.arange(n_out, dtype=np.float64) * (n_in - 1) / (n_out - 1)
    left = np.minimum(pos.astype(np.int64), n_in - 2)
    frac = pos - left
    m = np.zeros((n_out, n_in), np.float64)
    m[np.arange(n_out), left] = 1.0 - frac
    m[np.arange(n_out), left + 1] += frac
    return m.astype(np.float32)


@functools.lru_cache(maxsize=None)
def _up2_matrix(h, w):
    return np.kron(_lin_interp(h, 2 * h), _lin_interp(w, 2 * w))


def _fpn_body(c5_ref, c4_ref, c3_ref, w4t_ref, b4_ref, w3t_ref, b3_ref,
              u1_ref, u2_ref, p3_ref, p4_ref, p5_ref):
    bf16, f32 = jnp.bfloat16, jnp.float32
    G, HW5, C5 = c5_ref.shape
    x5 = c5_ref[...].reshape(G * HW5, C5).astype(bf16)
    y5 = jnp.dot(x5, w4t_ref[...], preferred_element_type=f32) + b4_ref[...]
    y5 = y5.astype(bf16).reshape(G, HW5, -1)
    for g in range(G):
        p5 = jnp.dot(u1_ref[...], y5[g], preferred_element_type=f32)
        p5_ref[g] = p5
        t = (p5 + c4_ref[g]).astype(bf16)
        y4 = jnp.dot(t, w3t_ref[...], preferred_element_type=f32) + b3_ref[...]
        p4 = jnp.dot(u2_ref[...], y4.astype(bf16), preferred_element_type=f32)
        p4_ref[g] = p4
        p3_ref[g] = p4 + c3_ref[g]


def kernel(w_c4, b_c4, w_c3, b_c3, c3, c4, c5):
    B, C5, H5, W5 = c5.shape
    C4 = w_c4.shape[0]
    C3 = w_c3.shape[0]
    HW5 = H5 * W5
    HW4 = 4 * HW5
    HW3 = 16 * HW5
    bf16 = jnp.bfloat16

    u1 = jnp.asarray(_up2_matrix(H5, W5), dtype=bf16)
    u2 = jnp.asarray(_up2_matrix(2 * H5, 2 * W5), dtype=bf16)

    c5r = jnp.transpose(c5, (0, 2, 3, 1)).reshape(B, HW5, C5)
    c4r = jnp.transpose(c4, (0, 2, 3, 1)).reshape(B, HW4, C4)
    c3r = jnp.transpose(c3, (0, 2, 3, 1)).reshape(B, HW3, C3)

    G = 8 if B % 8 == 0 else (2 if B % 2 == 0 else 1)
    batch_spec = lambda r, c: pl.BlockSpec((G, r, c), lambda i: (i, 0, 0))
    fixed_spec = lambda r, c: pl.BlockSpec((r, c), lambda i: (0, 0))
    in_specs = [
        batch_spec(HW5, C5),
        batch_spec(HW4, C4),
        batch_spec(HW3, C3),
        fixed_spec(C5, C4), fixed_spec(1, C4),
        fixed_spec(C4, C3), fixed_spec(1, C3),
        fixed_spec(HW4, HW5), fixed_spec(HW3, HW4),
    ]
    p3, p4, p5 = pl.pallas_call(
        _fpn_body,
        grid=(B // G,),
        in_specs=in_specs,
        out_specs=(batch_spec(HW3, C3), batch_spec(HW3, C3),
                   batch_spec(HW4, C4)),
        out_shape=(jax.ShapeDtypeStruct((B, HW3, C3), c3.dtype),
                   jax.ShapeDtypeStruct((B, HW3, C3), c3.dtype),
                   jax.ShapeDtypeStruct((B, HW4, C4), c4.dtype)),
        compiler_params=pltpu.CompilerParams(
            dimension_semantics=("parallel",),
            vmem_limit_bytes=64 << 20),
    )(c5r, c4r, c3r,
      w_c4.T.astype(bf16), b_c4.reshape(1, C4),
      w_c3.T.astype(bf16), b_c3.reshape(1, C3),
      u1, u2)

    def _to_nchw(x, c, hw):
        h = int(round(hw ** 0.5))
        return jnp.transpose(x.reshape(B, h, h, c), (0, 3, 1, 2))

    return (_to_nchw(p3, C3, HW3), _to_nchw(p4, C3, HW3),
            _to_nchw(p5, C4, HW4))


# speedup vs baseline: 4.3394x; 1.0077x over previous
import functools

import numpy as np
import jax
import jax.numpy as jnp
from jax.experimental import pallas as pl
from jax.experimental.pallas import tpu as pltpu


def _lin_interp(n_in, n_out):
    if n_in == 1:
        return np.ones((n_out, 1), np.float32)
    pos = np.arange(n_out, dtype=np.float64) * (n_in - 1) / (n_out - 1)
    left = np.minimum(pos.astype(np.int64), n_in - 2)
    frac = pos - left
    m = np.zeros((n_out, n_in), np.float64)
    m[np.arange(n_out), left] = 1.0 - frac
    m[np.arange(n_out), left + 1] += frac
    return m.astype(np.float32)


@functools.lru_cache(maxsize=None)
def _up2_matrix(h, w):
    return np.kron(_lin_interp(h, 2 * h), _lin_interp(w, 2 * w))


def _fpn_body(c5_ref, c4_ref, c3_ref, w4t_ref, b4_ref, w3t_ref, b3_ref,
              u1_ref, u2_ref, p3_ref, p4_ref, p5_ref):
    bf16, f32 = jnp.bfloat16, jnp.float32
    G, HW5, C5 = c5_ref.shape
    x5 = c5_ref[...].reshape(G * HW5, C5).astype(bf16)
    y5 = jnp.dot(x5, w4t_ref[...], preferred_element_type=f32) + b4_ref[...]
    y5 = y5.astype(bf16).reshape(G, HW5, -1)
    for g in range(G):
        p5 = jnp.dot(u1_ref[...], y5[g], preferred_element_type=f32)
        p5_ref[g] = p5
        t = (p5 + c4_ref[g]).astype(bf16)
        y4 = jnp.dot(t, w3t_ref[...], preferred_element_type=f32) + b3_ref[...]
        p4 = jnp.dot(u2_ref[...], y4.astype(bf16), preferred_element_type=f32)
        p4_ref[g] = p4
        p3_ref[g] = p4 + c3_ref[g]


def kernel(w_c4, b_c4, w_c3, b_c3, c3, c4, c5):
    B, C5, H5, W5 = c5.shape
    C4 = w_c4.shape[0]
    C3 = w_c3.shape[0]
    HW5 = H5 * W5
    HW4 = 4 * HW5
    HW3 = 16 * HW5
    bf16 = jnp.bfloat16

    u1 = jnp.asarray(_up2_matrix(H5, W5), dtype=bf16)
    u2 = jnp.asarray(_up2_matrix(2 * H5, 2 * W5), dtype=bf16)

    c5r = jnp.transpose(c5, (0, 2, 3, 1)).reshape(B, HW5, C5)
    c4r = jnp.transpose(c4, (0, 2, 3, 1)).reshape(B, HW4, C4)
    c3r = jnp.transpose(c3, (0, 2, 3, 1)).reshape(B, HW3, C3)

    G = next(g for g in (8, 4, 2, 1) if B % g == 0)
    batch_spec = lambda r, c: pl.BlockSpec((G, r, c), lambda i: (i, 0, 0))
    fixed_spec = lambda r, c: pl.BlockSpec((r, c), lambda i: (0, 0))
    in_specs = [
        batch_spec(HW5, C5),
        batch_spec(HW4, C4),
        batch_spec(HW3, C3),
        fixed_spec(C5, C4), fixed_spec(1, C4),
        fixed_spec(C4, C3), fixed_spec(1, C3),
        fixed_spec(HW4, HW5), fixed_spec(HW3, HW4),
    ]
    p3, p4, p5 = pl.pallas_call(
        _fpn_body,
        grid=(B // G,),
        in_specs=in_specs,
        out_specs=(batch_spec(HW3, C3), batch_spec(HW3, C3),
                   batch_spec(HW4, C4)),
        out_shape=(jax.ShapeDtypeStruct((B, HW3, C3), c3.dtype),
                   jax.ShapeDtypeStruct((B, HW3, C3), c3.dtype),
                   jax.ShapeDtypeStruct((B, HW4, C4), c4.dtype)),
        compiler_params=pltpu.CompilerParams(
            dimension_semantics=("parallel",),
            vmem_limit_bytes=64 << 20),
    )(c5r, c4r, c3r,
      w_c4.T.astype(bf16), b_c4.reshape(1, C4),
      w_c3.T.astype(bf16), b_c3.reshape(1, C3),
      u1, u2)

    def _to_nchw(x, c, h, w):
        return jnp.transpose(x.reshape(B, h, w, c), (0, 3, 1, 2))

    return (_to_nchw(p3, C3, 4 * H5, 4 * W5), _to_nchw(p4, C3, 4 * H5, 4 * W5),
            _to_nchw(p5, C4, 2 * H5, 2 * W5))
